# Initial kernel scaffold; baseline (speedup 1.0000x reference)
import jax
import jax.numpy as jnp
from jax.experimental import pallas as pl


def kernel(patches):
    raise NotImplementedError("write your pallas kernel here")


# speedup vs baseline: 36.7413x; 36.7413x over previous
import functools

import jax
import jax.numpy as jnp
from jax import lax
from jax.experimental import pallas as pl
from jax.experimental.pallas import tpu as pltpu
from jax.experimental.pallas import tpu_sc as plsc

_B = 256
_H = 256
_W = 256
_PIX = _H * _W
_NBINS = 256

_NC = 2
_NS = 16
_NW = _NC * _NS
_PPW = _B // _NW
_CHUNK = 8192
_NCHUNK = _PIX // _CHUNK
_L = 16


def _hist_body(p_hbm, out_hbm, buf, bank, stage):
    wid = lax.axis_index("s") * _NC + lax.axis_index("c")
    lane = lax.iota(jnp.int32, _L)
    lane_off = lane * _NBINS
    ones = jnp.ones((_L,), jnp.float32)
    zeros16 = jnp.zeros((_L,), jnp.float32)

    def per_patch(pi, carry):
        patch = wid * _PPW + pi

        def zero_body(i, c):
            bank[pl.ds(i * _L, _L)] = zeros16
            return c

        lax.fori_loop(0, (_NBINS * _L) // _L, zero_body, 0)

        def per_chunk(g, c):
            pltpu.sync_copy(p_hbm.at[patch, pl.ds(g * _CHUNK, _CHUNK)],
                            buf)

            def per_vec(i, cc):
                x = buf[pl.ds(i * _L, _L)]
                q = (jnp.clip(x, 0.0, 1.0) * 255.0).astype(jnp.int32)
                plsc.addupdate_scatter(bank, [q + lane_off], ones)
                return cc

            lax.fori_loop(0, _CHUNK // _L, per_vec, 0)
            return c

        lax.fori_loop(0, _NCHUNK, per_chunk, 0)

        def per_binchunk(cidx, c):
            acc = bank[pl.ds(cidx * _L, _L)]
            for l in range(1, _L):
                acc = acc + bank[pl.ds(l * _NBINS + cidx * _L, _L)]
            stage[pl.ds(cidx * _L, _L)] = acc
            return c

        lax.fori_loop(0, _NBINS // _L, per_binchunk, 0)
        pltpu.sync_copy(stage, out_hbm.at[patch])
        return carry

    lax.fori_loop(0, _PPW, per_patch, 0)


@functools.lru_cache(maxsize=None)
def _make_hist_call():
    mesh = plsc.VectorSubcoreMesh(core_axis_name="c", subcore_axis_name="s",
                                  num_cores=_NC, num_subcores=_NS)
    return functools.partial(
        pl.kernel,
        out_type=jax.ShapeDtypeStruct((_B, _NBINS), jnp.float32),
        mesh=mesh,
        compiler_params=pltpu.CompilerParams(needs_layout_passes=False),
        scratch_types=[
            pltpu.VMEM((_CHUNK,), jnp.float32),
            pltpu.VMEM((_NBINS * _L,), jnp.float32),
            pltpu.VMEM((_NBINS,), jnp.float32),
        ],
    )(_hist_body)


def _cues_body(x_ref, var_ref, edge_ref):
    x = x_ref[0]
    n = _PIX
    mean = jnp.sum(x) / n
    d = x - mean
    var = jnp.sum(d * d) * (1.0 / (n - 1))

    zrow = jnp.zeros((1, _W), x.dtype)
    zcol = jnp.zeros((_H, 1), x.dtype)
    up = jnp.concatenate([x[1:, :], zrow], axis=0)
    dn = jnp.concatenate([zrow, x[:-1, :]], axis=0)
    v = dn + 2.0 * x + up
    lf = jnp.concatenate([x[:, 1:], zcol], axis=1)
    rt = jnp.concatenate([zcol, x[:, :-1]], axis=1)
    h = lf + 2.0 * x + rt
    sx = (jnp.concatenate([v[:, 1:], zcol], axis=1)
          - jnp.concatenate([zcol, v[:, :-1]], axis=1))
    sy = (jnp.concatenate([h[1:, :], zrow], axis=0)
          - jnp.concatenate([zrow, h[:-1, :]], axis=0))
    edge = jnp.maximum(jnp.max(jnp.abs(sx)), jnp.max(jnp.abs(sy)))

    var_ref[...] = jnp.full((1, 1, 128), var, x.dtype)
    edge_ref[...] = jnp.full((1, 1, 128), edge, x.dtype)


_cues_call = pl.pallas_call(
    _cues_body,
    grid=(_B,),
    in_specs=[pl.BlockSpec((1, _H, _W), lambda i: (i, 0, 0))],
    out_specs=[pl.BlockSpec((1, 1, 128), lambda i: (i, 0, 0)),
               pl.BlockSpec((1, 1, 128), lambda i: (i, 0, 0))],
    out_shape=[jax.ShapeDtypeStruct((_B, 1, 128), jnp.float32),
               jax.ShapeDtypeStruct((_B, 1, 128), jnp.float32)],
)


def _entropy_body(h_ref, out_ref):
    hist = h_ref[...]
    p = hist * (1.0 / _PIX)
    plog = p * jnp.log2(jnp.maximum(p, 1e-8))
    ent = -jnp.sum(plog, axis=1)
    out_ref[...] = jnp.broadcast_to(ent[:, None], (_B, 128))


_entropy_call = pl.pallas_call(
    _entropy_body,
    out_shape=jax.ShapeDtypeStruct((_B, 128), jnp.float32),
)


def kernel(patches):
    flat = patches.reshape(_B, _PIX)
    hist = _make_hist_call()(flat)
    var, edge = _cues_call(patches.reshape(_B, _H, _W))
    ent = _entropy_call(hist)
    mahalanobis = jnp.zeros((_B,), patches.dtype)
    return (var[:, 0, 0], ent[:, 0], edge[:, 0, 0], mahalanobis)


# speedup vs baseline: 49.5109x; 1.3476x over previous
import functools

import jax
import jax.numpy as jnp
from jax import lax
from jax.experimental import pallas as pl
from jax.experimental.pallas import tpu as pltpu
from jax.experimental.pallas import tpu_sc as plsc

_B = 256
_H = 256
_W = 256
_PIX = _H * _W
_NBINS = 256

_NC = 2
_NS = 16
_NW = _NC * _NS
_PPW = _B // _NW
_CROWS = 32
_CHUNKS_PER_PATCH = _H // _CROWS
_NCHUNK = _PPW * _CHUNKS_PER_PATCH
_L = 16


def _hist_body(p_hbm, out_hbm, buf, bank, stage, sem0, sem1):
    wid = lax.axis_index("s") * _NC + lax.axis_index("c")
    lane_off = lax.iota(jnp.int32, _L) * _NBINS
    ones = jnp.ones((_L,), jnp.float32)
    zeros16 = jnp.zeros((_L,), jnp.float32)
    patch0 = wid * _PPW

    def chunk_src(g):
        return p_hbm.at[patch0 + g // _CHUNKS_PER_PATCH, 0,
                        pl.ds((g % _CHUNKS_PER_PATCH) * _CROWS, _CROWS)]

    def start(g, slot):
        pltpu.make_async_copy(chunk_src(g), buf.at[slot],
                              sem0 if slot == 0 else sem1).start()

    def wait(g, slot):
        pltpu.make_async_copy(chunk_src(g), buf.at[slot],
                              sem0 if slot == 0 else sem1).wait()

    def process(slot):
        def row_body(r, c):
            for u in range(_W // _L):
                x = buf[slot, r, pl.ds(u * _L, _L)]
                q = (jnp.clip(x, 0.0, 1.0) * 255.0).astype(jnp.int32)
                plsc.addupdate_scatter(bank, [q + lane_off], ones)
            return c

        lax.fori_loop(0, _CROWS, row_body, 0)

    def zero_bank():
        def zb(i, c):
            bank[pl.ds(i * _L, _L)] = zeros16
            return c

        lax.fori_loop(0, (_NBINS * _L) // _L, zb, 0)

    def finalize(patch):
        def red(cidx, c):
            acc = bank[pl.ds(cidx * _L, _L)]
            bank[pl.ds(cidx * _L, _L)] = zeros16
            for l in range(1, _L):
                off = l * _NBINS + cidx * _L
                acc = acc + bank[pl.ds(off, _L)]
                bank[pl.ds(off, _L)] = zeros16
            stage[pl.ds(cidx * _L, _L)] = acc
            return c

        lax.fori_loop(0, _NBINS // _L, red, 0)
        pltpu.sync_copy(stage, out_hbm.at[patch])

    zero_bank()
    start(0, 0)

    def pair_body(j, c):
        g0 = 2 * j
        g1 = g0 + 1
        start(g1, 1)
        wait(g0, 0)
        process(0)

        @pl.when(g0 + 2 < _NCHUNK)
        def _():
            start(g0 + 2, 0)

        wait(g1, 1)
        process(1)

        @pl.when(g1 % _CHUNKS_PER_PATCH == _CHUNKS_PER_PATCH - 1)
        def _():
            finalize(patch0 + g1 // _CHUNKS_PER_PATCH)

        return c

    lax.fori_loop(0, _NCHUNK // 2, pair_body, 0)


@functools.lru_cache(maxsize=None)
def _make_hist_call():
    mesh = plsc.VectorSubcoreMesh(core_axis_name="c", subcore_axis_name="s",
                                  num_cores=_NC, num_subcores=_NS)
    return functools.partial(
        pl.kernel,
        out_type=jax.ShapeDtypeStruct((_B, _NBINS), jnp.float32),
        mesh=mesh,
        compiler_params=pltpu.CompilerParams(needs_layout_passes=False),
        scratch_types=[
            pltpu.VMEM((2, _CROWS, _W), jnp.float32),
            pltpu.VMEM((_NBINS * _L,), jnp.float32),
            pltpu.VMEM((_NBINS,), jnp.float32),
            pltpu.SemaphoreType.DMA,
            pltpu.SemaphoreType.DMA,
        ],
    )(_hist_body)


def _cues_body(x_ref, var_ref, edge_ref):
    x = x_ref[0, 0]
    n = _PIX
    mean = jnp.sum(x) / n
    d = x - mean
    var = jnp.sum(d * d) * (1.0 / (n - 1))

    zrow = jnp.zeros((1, _W), x.dtype)
    zcol = jnp.zeros((_H, 1), x.dtype)
    up = jnp.concatenate([x[1:, :], zrow], axis=0)
    dn = jnp.concatenate([zrow, x[:-1, :]], axis=0)
    v = dn + 2.0 * x + up
    lf = jnp.concatenate([x[:, 1:], zcol], axis=1)
    rt = jnp.concatenate([zcol, x[:, :-1]], axis=1)
    h = lf + 2.0 * x + rt
    sx = (jnp.concatenate([v[:, 1:], zcol], axis=1)
          - jnp.concatenate([zcol, v[:, :-1]], axis=1))
    sy = (jnp.concatenate([h[1:, :], zrow], axis=0)
          - jnp.concatenate([zrow, h[:-1, :]], axis=0))
    edge = jnp.maximum(jnp.max(jnp.abs(sx)), jnp.max(jnp.abs(sy)))

    var_ref[...] = jnp.full((1, 1, 128), var, x.dtype)
    edge_ref[...] = jnp.full((1, 1, 128), edge, x.dtype)


_cues_call = pl.pallas_call(
    _cues_body,
    grid=(_B,),
    in_specs=[pl.BlockSpec((1, 1, _H, _W), lambda i: (i, 0, 0, 0))],
    out_specs=[pl.BlockSpec((1, 1, 128), lambda i: (i, 0, 0)),
               pl.BlockSpec((1, 1, 128), lambda i: (i, 0, 0))],
    out_shape=[jax.ShapeDtypeStruct((_B, 1, 128), jnp.float32),
               jax.ShapeDtypeStruct((_B, 1, 128), jnp.float32)],
)


def _entropy_body(h_ref, out_ref):
    hist = h_ref[...]
    p = hist * (1.0 / _PIX)
    plog = p * jnp.log2(jnp.maximum(p, 1e-8))
    ent = -jnp.sum(plog, axis=1)
    out_ref[...] = jnp.broadcast_to(ent[:, None], (_B, 128))


_entropy_call = pl.pallas_call(
    _entropy_body,
    out_shape=jax.ShapeDtypeStruct((_B, 128), jnp.float32),
)


def kernel(patches):
    hist = _make_hist_call()(patches)
    var, edge = _cues_call(patches)
    ent = _entropy_call(hist)
    mahalanobis = jnp.zeros((_B,), patches.dtype)
    return (var[:, 0, 0], ent[:, 0], edge[:, 0, 0], mahalanobis)


# speedup vs baseline: 91.7542x; 1.8532x over previous
import functools

import jax
import jax.numpy as jnp
from jax import lax
from jax.experimental import pallas as pl
from jax.experimental.pallas import tpu as pltpu
from jax.experimental.pallas import tpu_sc as plsc

_B = 256
_H = 256
_W = 256
_PIX = _H * _W
_NBINS = 256

_NC = 2
_NS = 16
_NW = _NC * _NS
_PPW = _B // _NW
_CROWS = 32
_CHUNKS_PER_PATCH = _H // _CROWS
_NCHUNK = _PPW * _CHUNKS_PER_PATCH
_L = 16


def _hist_body(p_hbm, out_hbm, buf, bank, stage, sem0, sem1):
    wid = lax.axis_index("s") * _NC + lax.axis_index("c")
    lane_off = lax.iota(jnp.int32, _L) * _NBINS
    ones = jnp.ones((_L,), jnp.float32)
    zeros16 = jnp.zeros((_L,), jnp.float32)
    patch0 = wid * _PPW

    def chunk_src(g):
        return p_hbm.at[patch0 + g // _CHUNKS_PER_PATCH, 0,
                        pl.ds((g % _CHUNKS_PER_PATCH) * _CROWS, _CROWS)]

    def start(g, slot):
        pltpu.make_async_copy(chunk_src(g), buf.at[slot],
                              sem0 if slot == 0 else sem1).start()

    def wait(g, slot):
        pltpu.make_async_copy(chunk_src(g), buf.at[slot],
                              sem0 if slot == 0 else sem1).wait()

    def process(slot):
        def row_body(r, c):
            xs = [buf[slot, r, pl.ds(u * _L, _L)] for u in range(_W // _L)]
            qs = [(jnp.clip(x, 0.0, 1.0) * 255.0).astype(jnp.int32)
                  + lane_off for x in xs]
            for q in qs:
                plsc.addupdate_scatter(bank, [q], ones)
            return c

        lax.fori_loop(0, _CROWS, row_body, 0)

    def zero_bank():
        def zb(i, c):
            bank[pl.ds(i * _L, _L)] = zeros16
            return c

        lax.fori_loop(0, (_NBINS * _L) // _L, zb, 0)

    def finalize(patch):
        def red(cidx, c):
            acc = bank[pl.ds(cidx * _L, _L)]
            bank[pl.ds(cidx * _L, _L)] = zeros16
            for l in range(1, _L):
                off = l * _NBINS + cidx * _L
                acc = acc + bank[pl.ds(off, _L)]
                bank[pl.ds(off, _L)] = zeros16
            stage[pl.ds(cidx * _L, _L)] = acc
            return c

        lax.fori_loop(0, _NBINS // _L, red, 0)
        pltpu.sync_copy(stage, out_hbm.at[patch])

    zero_bank()
    start(0, 0)

    def pair_body(j, c):
        g0 = 2 * j
        g1 = g0 + 1
        start(g1, 1)
        wait(g0, 0)
        process(0)

        @pl.when(g0 + 2 < _NCHUNK)
        def _():
            start(g0 + 2, 0)

        wait(g1, 1)
        process(1)

        @pl.when(g1 % _CHUNKS_PER_PATCH == _CHUNKS_PER_PATCH - 1)
        def _():
            finalize(patch0 + g1 // _CHUNKS_PER_PATCH)

        return c

    lax.fori_loop(0, _NCHUNK // 2, pair_body, 0)


@functools.lru_cache(maxsize=None)
def _make_hist_call():
    mesh = plsc.VectorSubcoreMesh(core_axis_name="c", subcore_axis_name="s",
                                  num_cores=_NC, num_subcores=_NS)
    return functools.partial(
        pl.kernel,
        out_type=jax.ShapeDtypeStruct((_B, _NBINS), jnp.float32),
        mesh=mesh,
        compiler_params=pltpu.CompilerParams(needs_layout_passes=False),
        scratch_types=[
            pltpu.VMEM((2, _CROWS, _W), jnp.float32),
            pltpu.VMEM((_NBINS * _L,), jnp.float32),
            pltpu.VMEM((_NBINS,), jnp.float32),
            pltpu.SemaphoreType.DMA,
            pltpu.SemaphoreType.DMA,
        ],
    )(_hist_body)


def _cues_body(x_ref, var_ref, edge_ref):
    x = x_ref[0, 0]
    n = _PIX
    mean = jnp.sum(x) / n
    d = x - mean
    var = jnp.sum(d * d) * (1.0 / (n - 1))

    zrow = jnp.zeros((1, _W), x.dtype)
    zcol = jnp.zeros((_H, 1), x.dtype)
    up = jnp.concatenate([x[1:, :], zrow], axis=0)
    dn = jnp.concatenate([zrow, x[:-1, :]], axis=0)
    v = dn + 2.0 * x + up
    lf = jnp.concatenate([x[:, 1:], zcol], axis=1)
    rt = jnp.concatenate([zcol, x[:, :-1]], axis=1)
    h = lf + 2.0 * x + rt
    sx = (jnp.concatenate([v[:, 1:], zcol], axis=1)
          - jnp.concatenate([zcol, v[:, :-1]], axis=1))
    sy = (jnp.concatenate([h[1:, :], zrow], axis=0)
          - jnp.concatenate([zrow, h[:-1, :]], axis=0))
    edge = jnp.maximum(jnp.max(jnp.abs(sx)), jnp.max(jnp.abs(sy)))

    var_ref[...] = jnp.full((1, 1, 128), var, x.dtype)
    edge_ref[...] = jnp.full((1, 1, 128), edge, x.dtype)


_cues_call = pl.pallas_call(
    _cues_body,
    grid=(_B,),
    in_specs=[pl.BlockSpec((1, 1, _H, _W), lambda i: (i, 0, 0, 0))],
    out_specs=[pl.BlockSpec((1, 1, 128), lambda i: (i, 0, 0)),
               pl.BlockSpec((1, 1, 128), lambda i: (i, 0, 0))],
    out_shape=[jax.ShapeDtypeStruct((_B, 1, 128), jnp.float32),
               jax.ShapeDtypeStruct((_B, 1, 128), jnp.float32)],
)


def _entropy_body(h_ref, out_ref):
    hist = h_ref[...]
    p = hist * (1.0 / _PIX)
    plog = p * jnp.log2(jnp.maximum(p, 1e-8))
    ent = -jnp.sum(plog, axis=1)
    out_ref[...] = jnp.broadcast_to(ent[:, None], (_B, 128))


_entropy_call = pl.pallas_call(
    _entropy_body,
    out_shape=jax.ShapeDtypeStruct((_B, 128), jnp.float32),
)


def kernel(patches):
    hist = _make_hist_call()(patches)
    var, edge = _cues_call(patches)
    ent = _entropy_call(hist)
    mahalanobis = jnp.zeros((_B,), patches.dtype)
    return (var[:, 0, 0], ent[:, 0], edge[:, 0, 0], mahalanobis)


# speedup vs baseline: 163.7191x; 1.7843x over previous
import functools

import jax
import jax.numpy as jnp
from jax import lax
from jax.experimental import pallas as pl
from jax.experimental.pallas import tpu as pltpu
from jax.experimental.pallas import tpu_sc as plsc

_B = 256
_H = 256
_W = 256
_PIX = _H * _W
_NBINS = 256

_NC = 2
_NS = 16
_NW = _NC * _NS
_PPW = _B // _NW
_CROWS = 32
_CHUNKS_PER_PATCH = _H // _CROWS
_NCHUNK = _PPW * _CHUNKS_PER_PATCH
_L = 16


def _hist_body(p_hbm, out_hbm, buf, bank, stage, sem0, sem1):
    wid = lax.axis_index("s") * _NC + lax.axis_index("c")
    lane_off = lax.iota(jnp.int32, _L) * _NBINS
    ones = jnp.ones((_L,), jnp.float32)
    zeros16 = jnp.zeros((_L,), jnp.float32)
    patch0 = wid * _PPW

    def chunk_src(g):
        return p_hbm.at[patch0 + g // _CHUNKS_PER_PATCH, 0,
                        pl.ds((g % _CHUNKS_PER_PATCH) * _CROWS, _CROWS)]

    def start(g, slot):
        pltpu.make_async_copy(chunk_src(g), buf.at[slot],
                              sem0 if slot == 0 else sem1).start()

    def wait(g, slot):
        pltpu.make_async_copy(chunk_src(g), buf.at[slot],
                              sem0 if slot == 0 else sem1).wait()

    def process(slot):
        def row_body(r, c):
            xs = [buf[slot, r, pl.ds(u * _L, _L)] for u in range(_W // _L)]
            qs = [(jnp.clip(x, 0.0, 1.0) * 255.0).astype(jnp.int32)
                  + lane_off for x in xs]
            for q in qs:
                plsc.addupdate_scatter(bank, [q], ones)
            return c

        lax.fori_loop(0, _CROWS, row_body, 0)

    def zero_bank():
        def zb(i, c):
            bank[pl.ds(i * _L, _L)] = zeros16
            return c

        lax.fori_loop(0, (_NBINS * _L) // _L, zb, 0)

    def finalize(patch):
        def red(cidx, c):
            acc = bank[pl.ds(cidx * _L, _L)]
            bank[pl.ds(cidx * _L, _L)] = zeros16
            for l in range(1, _L):
                off = l * _NBINS + cidx * _L
                acc = acc + bank[pl.ds(off, _L)]
                bank[pl.ds(off, _L)] = zeros16
            stage[pl.ds(cidx * _L, _L)] = acc
            return c

        lax.fori_loop(0, _NBINS // _L, red, 0)
        pltpu.sync_copy(stage, out_hbm.at[patch])

    zero_bank()
    start(0, 0)

    def pair_body(j, c):
        g0 = 2 * j
        g1 = g0 + 1
        start(g1, 1)
        wait(g0, 0)
        process(0)

        @pl.when(g0 + 2 < _NCHUNK)
        def _():
            start(g0 + 2, 0)

        wait(g1, 1)
        process(1)

        @pl.when(g1 % _CHUNKS_PER_PATCH == _CHUNKS_PER_PATCH - 1)
        def _():
            finalize(patch0 + g1 // _CHUNKS_PER_PATCH)

        return c

    lax.fori_loop(0, _NCHUNK // 2, pair_body, 0)


@functools.lru_cache(maxsize=None)
def _make_hist_call():
    mesh = plsc.VectorSubcoreMesh(core_axis_name="c", subcore_axis_name="s",
                                  num_cores=_NC, num_subcores=_NS)
    return functools.partial(
        pl.kernel,
        out_type=jax.ShapeDtypeStruct((_B, _NBINS), jnp.float32),
        mesh=mesh,
        compiler_params=pltpu.CompilerParams(needs_layout_passes=False),
        scratch_types=[
            pltpu.VMEM((2, _CROWS, _W), jnp.float32),
            pltpu.VMEM((_NBINS * _L,), jnp.float32),
            pltpu.VMEM((_NBINS,), jnp.float32),
            pltpu.SemaphoreType.DMA,
            pltpu.SemaphoreType.DMA,
        ],
    )(_hist_body)


def _cues_body(x_ref, var_ref, edge_ref):
    for p in range(_PPS):
        _cues_one(x_ref[p, 0], p, var_ref, edge_ref)


def _cues_one(x, p, var_ref, edge_ref):
    n = _PIX
    s1 = jnp.sum(x)
    s2 = jnp.sum(x * x)
    var = (s2 - s1 * s1 * (1.0 / n)) * (1.0 / (n - 1))

    zrow = jnp.zeros((1, _W), x.dtype)
    zcol = jnp.zeros((_H, 1), x.dtype)
    up = jnp.concatenate([x[1:, :], zrow], axis=0)
    dn = jnp.concatenate([zrow, x[:-1, :]], axis=0)
    v = dn + 2.0 * x + up
    g = up - dn
    sx = (jnp.concatenate([v[:, 1:], zcol], axis=1)
          - jnp.concatenate([zcol, v[:, :-1]], axis=1))
    gl = jnp.concatenate([g[:, 1:], zcol], axis=1)
    gr = jnp.concatenate([zcol, g[:, :-1]], axis=1)
    sy = gl + 2.0 * g + gr
    edge = jnp.maximum(jnp.max(jnp.abs(sx)), jnp.max(jnp.abs(sy)))

    var_ref[p, 0, :] = jnp.full((128,), var, x.dtype)
    edge_ref[p, 0, :] = jnp.full((128,), edge, x.dtype)


_PPS = 8

_cues_call = pl.pallas_call(
    _cues_body,
    grid=(_B // _PPS,),
    in_specs=[pl.BlockSpec((_PPS, 1, _H, _W), lambda i: (i, 0, 0, 0))],
    out_specs=[pl.BlockSpec((_PPS, 1, 128), lambda i: (i, 0, 0)),
               pl.BlockSpec((_PPS, 1, 128), lambda i: (i, 0, 0))],
    out_shape=[jax.ShapeDtypeStruct((_B, 1, 128), jnp.float32),
               jax.ShapeDtypeStruct((_B, 1, 128), jnp.float32)],
)


def _entropy_body(h_ref, out_ref):
    hist = h_ref[...]
    p = hist * (1.0 / _PIX)
    plog = p * jnp.log2(jnp.maximum(p, 1e-8))
    ent = -jnp.sum(plog, axis=1)
    out_ref[...] = jnp.broadcast_to(ent[:, None], (_B, 128))


_entropy_call = pl.pallas_call(
    _entropy_body,
    out_shape=jax.ShapeDtypeStruct((_B, 128), jnp.float32),
)


def kernel(patches):
    hist = _make_hist_call()(patches)
    var, edge = _cues_call(patches)
    ent = _entropy_call(hist)
    mahalanobis = jnp.zeros((_B,), patches.dtype)
    return (var[:, 0, 0], ent[:, 0], edge[:, 0, 0], mahalanobis)
